# Initial kernel scaffold; baseline (speedup 1.0000x reference)
import jax
import jax.numpy as jnp
from jax.experimental import pallas as pl


def kernel(x, edge_index, W1, b1, W2, A2a, A2b, W3, A3a, A3b, W4, A4a, A4b):
    raise NotImplementedError("write your pallas kernel here")


# speedup vs baseline: 8.2762x; 8.2762x over previous
import functools

import jax
import jax.numpy as jnp
from jax import lax
from jax.experimental import pallas as pl
from jax.experimental.pallas import tpu as pltpu
from jax.experimental.pallas import tpu_sc as plsc

_N = 10000
_E = 320000
_H = 64

_NC = 2
_NS = 16
_NW = _NC * _NS
_NPAD = 10240
_RPT = _NPAD // _NS
_ZC = 32
_W = 128
_EB = 80
_EW = _E // _NW
_NCH = _EW // _EB

def _deg_body(dst_hbm, out_hbm, acc_sh, dst_v, ones_v, zero_v):
    cid = lax.axis_index("c")
    sid = lax.axis_index("s")
    for j in range(_EB // 16):
        ones_v[pl.ds(j * 16, 16)] = jnp.ones((16,), jnp.float32)
    for j in range(_ZC // 16):
        zero_v[pl.ds(j * 16, 16)] = jnp.zeros((16,), jnp.float32)
    rbase = sid * _RPT

    def zloop(k, car):
        pltpu.sync_copy(zero_v, acc_sh.at[pl.ds(rbase + k * _ZC, _ZC)])
        return car

    lax.fori_loop(0, _RPT // _ZC, zloop, 0)
    plsc.subcore_barrier()

    ebase = (cid * _NS + sid) * _EW

    def eloop(k, car):
        pltpu.sync_copy(dst_hbm.at[pl.ds(ebase + k * _EB, _EB)], dst_v)
        pltpu.sync_copy(ones_v, acc_sh.at[dst_v], add=True)
        return car

    lax.fori_loop(0, _NCH, eloop, 0)
    plsc.subcore_barrier()
    pltpu.sync_copy(acc_sh.at[pl.ds(rbase, _RPT)],
                    out_hbm.at[cid, pl.ds(rbase, _RPT)])


@functools.cache
def _get_deg_kernel():
    mesh = plsc.VectorSubcoreMesh(core_axis_name="c", subcore_axis_name="s")
    return pl.kernel(
        _deg_body,
        out_type=jax.ShapeDtypeStruct((_NC, _NPAD), jnp.float32),
        mesh=mesh,
        scratch_types=[
            pltpu.VMEM_SHARED((_NPAD,), jnp.float32),
            pltpu.VMEM((_EB,), jnp.int32),
            pltpu.VMEM((_EB,), jnp.float32),
            pltpu.VMEM((_ZC,), jnp.float32),
        ],
    )


def _conv_body(g_hbm, src_hbm, dst_hbm, out_hbm,
               acc_sh, src_v, dst_v, rows_v, zero_v, sem):
    cid = lax.axis_index("c")
    sid = lax.axis_index("s")
    for r in range(_ZC):
        for j in range(_W // 16):
            zero_v[r, pl.ds(j * 16, 16)] = jnp.zeros((16,), jnp.float32)
    rbase = sid * _RPT

    def zloop(k, car):
        pltpu.sync_copy(zero_v, acc_sh.at[pl.ds(rbase + k * _ZC, _ZC)])
        return car

    lax.fori_loop(0, _RPT // _ZC, zloop, 0)
    plsc.subcore_barrier()

    ebase = (cid * _NS + sid) * _EW

    def eloop(k, car):
        pltpu.sync_copy(src_hbm.at[pl.ds(ebase + k * _EB, _EB)], src_v)
        pltpu.sync_copy(dst_hbm.at[pl.ds(ebase + k * _EB, _EB)], dst_v)
        pltpu.async_copy(g_hbm.at[src_v], rows_v, sem).wait()
        pltpu.sync_copy(rows_v, acc_sh.at[dst_v], add=True)
        return car

    lax.fori_loop(0, _NCH, eloop, 0)
    plsc.subcore_barrier()
    pltpu.sync_copy(acc_sh.at[pl.ds(rbase, _RPT)],
                    out_hbm.at[cid, pl.ds(rbase, _RPT)])


@functools.cache
def _get_conv_kernel():
    mesh = plsc.VectorSubcoreMesh(core_axis_name="c", subcore_axis_name="s")
    return pl.kernel(
        _conv_body,
        out_type=jax.ShapeDtypeStruct((_NC, _NPAD, _W), jnp.float32),
        mesh=mesh,
        scratch_types=[
            pltpu.VMEM_SHARED((_NPAD, _W), jnp.float32),
            pltpu.VMEM((_EB,), jnp.int32),
            pltpu.VMEM((_EB,), jnp.int32),
            pltpu.VMEM((_EB, _W), jnp.float32),
            pltpu.VMEM((_ZC, _W), jnp.float32),
            pltpu.SemaphoreType.DMA,
        ],
    )

_RB = 1000
_GRID = _N // _RB


def _dinv_from(degp_ref):
    deg = degp_ref[0] + degp_ref[1]
    return jnp.where(deg > 0, lax.rsqrt(jnp.maximum(deg, 1.0)), 0.0)


def _stage1_body(x_ref, w1_ref, b1_ref, degp_ref, h_ref, g_ref):
    h = jnp.maximum(
        jnp.dot(x_ref[...], w1_ref[...], preferred_element_type=jnp.float32)
        + b1_ref[...], 0.0)
    dinv = _dinv_from(degp_ref)
    h_ref[...] = h
    g_ref[...] = jnp.pad(dinv * h, ((0, 0), (0, _W - _H)))


_stage1 = pl.pallas_call(
    _stage1_body,
    grid=(_GRID,),
    in_specs=[
        pl.BlockSpec((_RB, 128), lambda i: (i, 0)),
        pl.BlockSpec((128, _H), lambda i: (0, 0)),
        pl.BlockSpec((1, _H), lambda i: (0, 0)),
        pl.BlockSpec((_NC, _RB, 1), lambda i: (0, i, 0)),
    ],
    out_specs=[
        pl.BlockSpec((_RB, _H), lambda i: (i, 0)),
        pl.BlockSpec((_RB, _W), lambda i: (i, 0)),
    ],
    out_shape=[
        jax.ShapeDtypeStruct((_N, _H), jnp.float32),
        jax.ShapeDtypeStruct((_N, _W), jnp.float32),
    ],
)


def _mid_body(h_ref, parts_ref, degp_ref, wt_ref, wb_ref, aa_ref, ab_ref,
              hn_ref, gn_ref):
    h = h_ref[...]
    dinv = _dinv_from(degp_ref)
    c = dinv * (parts_ref[0, :, :_H] + parts_ref[1, :, :_H])
    z = (jnp.dot(h, wt_ref[...], preferred_element_type=jnp.float32)
         + jnp.dot(c, wb_ref[...], preferred_element_type=jnp.float32)
         + jnp.dot(h, aa_ref[...], preferred_element_type=jnp.float32)
         * jnp.dot(h, ab_ref[...], preferred_element_type=jnp.float32))
    hn = jnp.maximum(z, 0.0)
    hn_ref[...] = hn
    gn_ref[...] = jnp.pad(dinv * hn, ((0, 0), (0, _W - _H)))


def _last_body(h_ref, parts_ref, degp_ref, wt_ref, wb_ref, aa_ref, ab_ref,
               out_ref):
    h = h_ref[...]
    dinv = _dinv_from(degp_ref)
    c = dinv * (parts_ref[0, :, :_H] + parts_ref[1, :, :_H])
    z = (jnp.dot(h, wt_ref[...], preferred_element_type=jnp.float32)
         + jnp.dot(c, wb_ref[...], preferred_element_type=jnp.float32)
         + jnp.dot(h, aa_ref[...], preferred_element_type=jnp.float32)
         * jnp.dot(h, ab_ref[...], preferred_element_type=jnp.float32))
    out_ref[...] = jnp.maximum(z, 0.0)


_mid_in_specs = [
    pl.BlockSpec((_RB, _H), lambda i: (i, 0)),
    pl.BlockSpec((_NC, _RB, _W), lambda i: (0, i, 0)),
    pl.BlockSpec((_NC, _RB, 1), lambda i: (0, i, 0)),
    pl.BlockSpec((_H, _H), lambda i: (0, 0)),
    pl.BlockSpec((_H, _H), lambda i: (0, 0)),
    pl.BlockSpec((_H, _H), lambda i: (0, 0)),
    pl.BlockSpec((_H, _H), lambda i: (0, 0)),
]

_stage_mid = pl.pallas_call(
    _mid_body,
    grid=(_GRID,),
    in_specs=_mid_in_specs,
    out_specs=[
        pl.BlockSpec((_RB, _H), lambda i: (i, 0)),
        pl.BlockSpec((_RB, _W), lambda i: (i, 0)),
    ],
    out_shape=[
        jax.ShapeDtypeStruct((_N, _H), jnp.float32),
        jax.ShapeDtypeStruct((_N, _W), jnp.float32),
    ],
)

_stage_last = pl.pallas_call(
    _last_body,
    grid=(_GRID,),
    in_specs=_mid_in_specs,
    out_specs=pl.BlockSpec((_RB, _H), lambda i: (i, 0)),
    out_shape=jax.ShapeDtypeStruct((_N, _H), jnp.float32),
)


def kernel(x, edge_index, W1, b1, W2, A2a, A2b, W3, A3a, A3b, W4, A4a, A4b):
    src = edge_index[0].astype(jnp.int32)
    dst = edge_index[1].astype(jnp.int32)
    deg_kernel = _get_deg_kernel()
    conv_kernel = _get_conv_kernel()
    deg_parts = deg_kernel(dst).reshape(_NC, _NPAD, 1)
    h1, g1 = _stage1(x, W1, b1.reshape(1, _H), deg_parts)
    p1 = conv_kernel(g1, src, dst)
    h2, g2 = _stage_mid(h1, p1, deg_parts, W2[:_H], W2[_H:], A2a, A2b)
    p2 = conv_kernel(g2, src, dst)
    h3, g3 = _stage_mid(h2, p2, deg_parts, W3[:_H], W3[_H:], A3a, A3b)
    p3 = conv_kernel(g3, src, dst)
    out = _stage_last(h3, p3, deg_parts, W4[:_H], W4[_H:], A4a, A4b)
    return out
